# baseline (speedup 1.0000x reference)
import functools

import jax
import jax.numpy as jnp
from jax import lax
from jax.experimental import pallas as pl
from jax.experimental.pallas import tpu as pltpu
from jax.experimental.pallas import tpu_sc as plsc

NC, NS, LANES = 2, 16, 16
NW = NC * NS
K_EDGE = 64
WP = 128
CH = 64


def _mm_bias(x, w, b):
    M, _ = x.shape
    N = w.shape[1]

    def body(x_ref, w_ref, b_ref, o_ref):
        o_ref[...] = jnp.dot(x_ref[...], w_ref[...],
                             preferred_element_type=jnp.float32) + b_ref[...]

    return pl.pallas_call(
        body, out_shape=jax.ShapeDtypeStruct((M, N), jnp.float32),
    )(x, w, b.reshape(1, N))


def _self_init(xl, xr, att, H, ea=None, we=None):
    n, W = xl.shape
    has_ea = ea is not None

    def body(*refs):
        if has_ea:
            xl_ref, xr_ref, att_ref, ea_ref, we_ref, o_ref = refs
        else:
            xl_ref, xr_ref, att_ref, o_ref = refs
        t = xl_ref[...] + xr_ref[...]
        if has_ea:
            xe_row = jnp.dot(jnp.mean(ea_ref[...], axis=0, keepdims=True),
                             we_ref[...], preferred_element_type=jnp.float32)
            t = t + xe_row
        t = jnp.maximum(t, 0.2 * t)
        u = t * att_ref[...]
        zpad = jnp.zeros((n, WP - CH - 1), jnp.float32)
        slabs = []
        for h in range(H):
            s = jnp.sum(u[:, h * CH:(h + 1) * CH], axis=1, keepdims=True)
            p = jnp.exp(s)
            slabs.append(jnp.concatenate(
                [p * xl_ref[:, h * CH:(h + 1) * CH], p, zpad], axis=1))
        if H == 1:
            slabs.append(jnp.zeros((n, WP), jnp.float32))
        o_ref[...] = jnp.stack(slabs, axis=0)

    args = [xl, xr, att.reshape(1, W)]
    if has_ea:
        args += [ea, we]
    return pl.pallas_call(
        body, out_shape=jax.ShapeDtypeStruct((2, n, WP), jnp.float32),
    )(*args)


def _combine(parts, Nd, post=None, ln_g=None, ln_b=None):
    out_w = (parts[0][2] * CH) if parts[0][3] else CH
    has_ln = post in ('ln', 'ln_elu')

    def body(*refs):
        refs = list(refs)
        o = None
        for (_, _, H, concat) in parts:
            p_ref, b_ref = refs.pop(0), refs.pop(0)
            if H == 2:
                outs = [p_ref[h, :Nd, :CH]
                        / (p_ref[h, :Nd, CH:CH + 1] + 1e-16)
                        for h in range(2)]
                v = jnp.concatenate(outs, axis=1) if concat else (
                    (outs[0] + outs[1]) * 0.5)
            else:
                acc = p_ref[0] + p_ref[1]
                v = acc[:Nd, :CH] / (acc[:Nd, CH:CH + 1] + 1e-16)
            v = v + b_ref[...]
            o = v if o is None else o + v
        if has_ln:
            g_ref, lb_ref = refs.pop(0), refs.pop(0)
            mu = jnp.mean(o)
            var = jnp.mean((o - mu) ** 2)
            o = (o - mu) / jnp.sqrt(var + 1e-5) * g_ref[...] + lb_ref[...]
        if post in ('elu', 'ln_elu'):
            o = jnp.where(o > 0, o, jnp.exp(o) - 1.0)
        refs[0][...] = o

    args = []
    for (partials, b, H, concat) in parts:
        args += [partials, b.reshape(1, out_w)]
    if has_ln:
        args += [ln_g.reshape(1, out_w), ln_b.reshape(1, out_w)]
    return pl.pallas_call(
        body, out_shape=jax.ShapeDtypeStruct((Nd, out_w), jnp.float32),
    )(*args)


def _elu(x, x2=None):
    two = x2 is not None

    def body(*refs):
        if two:
            x_ref, y_ref, o_ref = refs
            v = x_ref[...] + y_ref[...]
        else:
            x_ref, o_ref = refs
            v = x_ref[...]
        o_ref[...] = jnp.where(v > 0, v, jnp.exp(v) - 1.0)

    args = [x] if not two else [x, x2]
    return pl.pallas_call(
        body, out_shape=jax.ShapeDtypeStruct(x.shape, jnp.float32),
    )(*args)


def _ln_graph(x, g, b, do_elu=False, x2=None):
    two = x2 is not None
    n, f = x.shape

    def body(*refs):
        if two:
            x_ref, y_ref, g_ref, b_ref, o_ref = refs
            v = x_ref[...] + y_ref[...]
        else:
            x_ref, g_ref, b_ref, o_ref = refs
            v = x_ref[...]
        mu = jnp.mean(v)
        var = jnp.mean((v - mu) ** 2)
        y = (v - mu) / jnp.sqrt(var + 1e-5) * g_ref[...] + b_ref[...]
        if do_elu:
            y = jnp.where(y > 0, y, jnp.exp(y) - 1.0)
        o_ref[...] = y

    args = ([x, x2] if two else [x]) + [g.reshape(1, f), b.reshape(1, f)]
    return pl.pallas_call(
        body, out_shape=jax.ShapeDtypeStruct((n, f), jnp.float32),
    )(*args)


def _scores(node_emb, targets_pad):
    M, Kd = node_emb.shape
    T = targets_pad.shape[0]
    BM = 2000

    def body(x_ref, t_ref, o_ref):
        o_ref[...] = lax.dot_general(
            x_ref[...], t_ref[...], (((1,), (1,)), ((), ())),
            preferred_element_type=jnp.float32) * 0.125

    return pl.pallas_call(
        body,
        grid=(M // BM,),
        in_specs=[pl.BlockSpec((BM, Kd), lambda i: (i, 0)),
                  pl.BlockSpec((T, Kd), lambda i: (0, 0))],
        out_specs=pl.BlockSpec((BM, T), lambda i: (i, 0)),
        out_shape=jax.ShapeDtypeStruct((M, T), jnp.float32),
    )(node_emb, targets_pad)


def _power_head(node_emb, veh_emb, pw1, pb1, pw2, pb2):
    def body(n_ref, v_ref, w1_ref, b1_ref, w2_ref, b2_ref, o_ref):
        ge = jnp.mean(n_ref[...], axis=0, keepdims=True)
        sv = jnp.mean(v_ref[...], axis=0, keepdims=True)
        pf = jnp.concatenate([ge, sv], axis=1)
        hp = jnp.maximum(
            jnp.dot(pf, w1_ref[...], preferred_element_type=jnp.float32)
            + b1_ref[...], 0.0)
        z = jnp.dot(hp, w2_ref[...],
                    preferred_element_type=jnp.float32) + b2_ref[...]
        o_ref[...] = 1.0 / (1.0 + jnp.exp(-z))

    return pl.pallas_call(
        body, out_shape=jax.ShapeDtypeStruct((1, 1), jnp.float32),
    )(node_emb, veh_emb, pw1, pb1.reshape(1, -1), pw2, pb2.reshape(1, -1))


@functools.lru_cache(maxsize=None)
def _edge_sweep_builder(E_pad, Nd_pad, H, has_xe):
    K = K_EDGE
    n_workers = NS if H == 2 else NW
    blocks_per_w = E_pad // (K * n_workers)
    assert blocks_per_w * K * n_workers == E_pad
    assert blocks_per_w % 2 == 0
    mesh = plsc.VectorSubcoreMesh(core_axis_name="c", subcore_axis_name="s",
                                  num_cores=NC, num_subcores=NS)

    def buf2(shape, dtype):
        return [pltpu.VMEM(shape, dtype), pltpu.VMEM(shape, dtype)]

    scratch = (
        buf2((K,), jnp.int32)
        + buf2((K,), jnp.int32)
        + buf2((K,), jnp.int32)
        + buf2((K, CH), jnp.float32)
        + buf2((K, CH), jnp.float32)
        + (buf2((K, CH), jnp.float32) if has_xe else [])
        + buf2((K, WP), jnp.float32)
        + [
            pltpu.VMEM((CH, 16), jnp.float32),
            pltpu.VMEM((2, CH, 16), jnp.float32),
            pltpu.VMEM_SHARED((Nd_pad, WP), jnp.float32),
            pltpu.SemaphoreType.DMA, pltpu.SemaphoreType.DMA,
            pltpu.SemaphoreType.DMA, pltpu.SemaphoreType.DMA,
        ]
    )

    @functools.partial(
        pl.kernel, mesh=mesh,
        compiler_params=pltpu.CompilerParams(needs_layout_passes=False,
                                             use_tc_tiling_on_sc=False),
        out_type=jax.ShapeDtypeStruct((NC, Nd_pad, WP), jnp.float32),
        scratch_types=scratch)
    def k(*refs):
        nx = 3 if has_xe else 2
        tabs = refs[:nx * H]
        src_hbm, dst_hbm = refs[nx * H], refs[nx * H + 1]
        atts = refs[nx * H + 2: nx * H + 2 + H]
        init_hbm = refs[nx * H + 2 + H]
        out_hbm = refs[nx * H + 3 + H]
        it = iter(refs[nx * H + 4 + H:])

        def take2():
            return [next(it), next(it)]

        src_v, dst_v, dsts_v = take2(), take2(), take2()
        xl_v, xr_v = take2(), take2()
        xe_v = take2() if has_xe else [None, None]
        out_v = take2()
        att_v, xlt_v, acc = next(it), next(it), next(it)
        sem_g, sem_s = take2(), take2()

        cid = lax.axis_index("c")
        sid = lax.axis_index("s")

        @pl.when(sid == 0)
        def _():
            pltpu.sync_copy(init_hbm.at[cid], acc)

        lanes = lax.iota(jnp.int32, 16)
        zeros16 = jnp.zeros((16,), jnp.float32)

        def zinit(i, c0):
            for cj in range(CH, WP, 16):
                out_v[0][i, pl.ds(cj, 16)] = zeros16
                out_v[1][i, pl.ds(cj, 16)] = zeros16
            return c0
        lax.fori_loop(0, K, zinit, 0)

        plsc.subcore_barrier()

        def run_edges(xl_hbm, xr_hbm, xe_hbm, att_hbm, worker):
            pltpu.sync_copy(att_hbm, att_v)
            wbase = worker * blocks_per_w

            def issue(b, g):
                base = pl.multiple_of((wbase + g) * K, K)
                pltpu.sync_copy(src_hbm.at[pl.ds(base, K)], src_v[b])
                pltpu.sync_copy(dst_hbm.at[pl.ds(base, K)], dst_v[b])
                pltpu.async_copy(xl_hbm.at[src_v[b]], xl_v[b], sem_g[b])
                pltpu.async_copy(xr_hbm.at[dst_v[b]], xr_v[b], sem_g[b])
                if xe_hbm is not None:
                    pltpu.async_copy(xe_hbm.at[pl.ds(base, K)], xe_v[b],
                                     sem_g[b])

            def compute(b, g):
                pltpu.make_async_copy(xl_hbm.at[src_v[b]], xl_v[b],
                                      sem_g[b]).wait()
                pltpu.make_async_copy(xr_hbm.at[dst_v[b]], xr_v[b],
                                      sem_g[b]).wait()
                if xe_hbm is not None:
                    pltpu.make_async_copy(xe_hbm.at[pl.ds(0, K)], xe_v[b],
                                          sem_g[b]).wait()

                @pl.when(g > 1)
                def _():
                    pltpu.make_async_copy(out_v[b], acc.at[dsts_v[b]],
                                          sem_s[b]).wait()

                @plsc.parallel_loop(0, K // 16, unroll=2)
                def group_body(g2):
                    rows = g2 * 16 + lanes
                    par = g2 & 1
                    lacc = [zeros16, zeros16, zeros16, zeros16]
                    for c in range(CH):
                        cvec = (lanes + c) & (CH - 1)
                        a = plsc.load_gather(xl_v[b], [rows, cvec])
                        t = a + plsc.load_gather(xr_v[b], [rows, cvec])
                        if xe_hbm is not None:
                            t = t + plsc.load_gather(xe_v[b], [rows, cvec])
                        t = jnp.maximum(t, 0.2 * t)
                        lacc[c % 4] = lacc[c % 4] + att_v[c, :] * t
                        xlt_v[par, c, :] = a
                    pv = jnp.exp((lacc[0] + lacc[1]) + (lacc[2] + lacc[3]))
                    for c in range(CH):
                        cvec = (lanes + c) & (CH - 1)
                        plsc.store_scatter(out_v[b], [rows, cvec],
                                           pv * xlt_v[par, c, :])
                    plsc.store_scatter(
                        out_v[b], [rows, jnp.full((16,), CH, jnp.int32)], pv)
                for q in range(K // 16):
                    dsts_v[b][pl.ds(q * 16, 16)] = dst_v[b][pl.ds(q * 16, 16)]
                pltpu.async_copy(out_v[b], acc.at[dsts_v[b]], sem_s[b],
                                 add=True)

            issue(0, 0)

            def pair_body(i, carry):
                g0 = i * 2
                issue(1, g0 + 1)
                compute(0, g0)

                @pl.when(i + 1 < blocks_per_w // 2)
                def _():
                    issue(0, g0 + 2)
                compute(1, g0 + 1)
                return carry

            lax.fori_loop(0, blocks_per_w // 2, pair_body, 0)
            pltpu.make_async_copy(out_v[0], acc.at[dsts_v[0]],
                                  sem_s[0]).wait()
            pltpu.make_async_copy(out_v[1], acc.at[dsts_v[1]],
                                  sem_s[1]).wait()

        if H == 2:
            @pl.when(cid == 0)
            def _():
                run_edges(tabs[0], tabs[1], tabs[2] if has_xe else None,
                          atts[0], sid)

            @pl.when(cid == 1)
            def _():
                run_edges(tabs[nx], tabs[nx + 1],
                          tabs[nx + 2] if has_xe else None, atts[1], sid)
        else:
            run_edges(tabs[0], tabs[1], tabs[2] if has_xe else None,
                      atts[0], sid * NC + cid)

        plsc.subcore_barrier()

        @pl.when(sid == 0)
        def _():
            pltpu.sync_copy(acc, out_hbm.at[cid])

    return k


def _round_up(x, m):
    return (x + m - 1) // m * m


def _pad_rows(x, rows):
    r = x.shape[0]
    if r < rows:
        x = jnp.concatenate(
            [x, jnp.zeros((rows - r,) + x.shape[1:], x.dtype)], axis=0)
    return x


def _gat_layer(x_src, x_dst, ei, ea, p, H, self_loops):
    W = H * CH
    Nd = x_dst.shape[0]
    Nd_pad = _round_up(Nd + 1, 16)
    E = ei.shape[1]
    E_pad = _round_up(E, K_EDGE * NW * 2)

    if x_src is x_dst:
        wlr = jnp.concatenate([p['Wl'], p['Wr']], axis=1)
        blr = jnp.concatenate([p['bl'], p['br']])
        xlr = _mm_bias(x_src, wlr, blr)
        xl, xr = xlr[:, :W], xlr[:, W:]
    else:
        xl = _mm_bias(x_src, p['Wl'], p['bl'])
        xr = _mm_bias(x_dst, p['Wr'], p['br'])
    att = p['att'].reshape(-1)

    src = jnp.concatenate(
        [ei[0], jnp.zeros((E_pad - E,), jnp.int32)]) if E_pad > E else ei[0]
    dst = jnp.concatenate(
        [ei[1], jnp.full((E_pad - E,), Nd, jnp.int32)]) if E_pad > E else ei[1]
    xr_pad = _pad_rows(xr, Nd_pad)

    if self_loops:
        init = _self_init(xl, xr, att, H, ea=ea, we=p.get('We'))
        init = jnp.concatenate(
            [init, jnp.zeros((2, Nd_pad - Nd, WP), jnp.float32)], axis=1)
    else:
        init = jnp.zeros((2, Nd_pad, WP), jnp.float32)

    has_xe = ea is not None
    if has_xe:
        ea_pad = (jnp.concatenate(
            [ea, jnp.zeros((E_pad - E, ea.shape[1]), jnp.float32)], axis=0)
            if E_pad > E else ea)
        xe = _mm_bias(ea_pad, p['We'], jnp.zeros((W,), jnp.float32))

    args = []
    for h in range(H):
        args.append(xl[:, h * CH:(h + 1) * CH])
        args.append(xr_pad[:, h * CH:(h + 1) * CH])
        if has_xe:
            args.append(xe[:, h * CH:(h + 1) * CH])
    args += [src, dst]
    diag = (jnp.arange(CH)[:, None] + jnp.arange(16)[None, :]) % CH
    for h in range(H):
        ah = att[h * CH:(h + 1) * CH]
        args.append(ah[diag])
    args.append(init)

    sweep = _edge_sweep_builder(E_pad, Nd_pad, H, has_xe)
    return sweep(*args)


def kernel(dag_x, dag_edge_index, veh_x, rsu_x, v2v_edge_index, v2v_edge_attr,
           v2i_edge_index, v2i_edge_attr, i2v_edge_index, i2v_edge_attr,
           params):
    P = params

    pg1 = _gat_layer(dag_x, dag_x, dag_edge_index, None, P['g1'], 2, True)
    h = _combine([(pg1, P['g1']['b'], 2, True)], dag_x.shape[0],
                 post='ln_elu', ln_g=P['ln1_g'], ln_b=P['ln1_b'])
    pg2 = _gat_layer(h, h, dag_edge_index, None, P['g2'], 1, True)
    node_emb = _combine([(pg2, P['g2']['b'], 1, False)], h.shape[0])

    p_v1 = _gat_layer(veh_x, veh_x, v2v_edge_index, v2v_edge_attr,
                      P['t1_v2v'], 2, True)
    p_r1 = _gat_layer(veh_x, rsu_x, v2i_edge_index, v2i_edge_attr,
                      P['t1_v2i'], 2, False)
    p_v1b = _gat_layer(rsu_x, veh_x, i2v_edge_index, i2v_edge_attr,
                       P['t1_i2v'], 2, False)
    veh_h = _combine([(p_v1, P['t1_v2v']['b'], 2, True),
                      (p_v1b, P['t1_i2v']['b'], 2, True)],
                     veh_x.shape[0], post='elu')
    rsu_h = _combine([(p_r1, P['t1_v2i']['b'], 2, True)],
                     rsu_x.shape[0], post='elu')

    p_v2 = _gat_layer(veh_h, veh_h, v2v_edge_index, v2v_edge_attr,
                      P['t2_v2v'], 1, True)
    p_r2 = _gat_layer(veh_h, rsu_h, v2i_edge_index, v2i_edge_attr,
                      P['t2_v2i'], 1, False)
    p_v2b = _gat_layer(rsu_h, veh_h, i2v_edge_index, i2v_edge_attr,
                       P['t2_i2v'], 1, False)
    veh_emb = _combine([(p_v2, P['t2_v2v']['b'], 1, False),
                        (p_v2b, P['t2_i2v']['b'], 1, False)],
                       veh_h.shape[0], post='ln',
                       ln_g=P['lnv_g'], ln_b=P['lnv_b'])
    rsu_emb = _combine([(p_r2, P['t2_v2i']['b'], 1, False)],
                       rsu_h.shape[0], post='ln',
                       ln_g=P['lnr_g'], ln_b=P['lnr_b'])

    targets = jnp.concatenate([rsu_emb, veh_emb], axis=0)
    T = targets.shape[0]
    T_pad = _round_up(T, 128)
    targets_pad = jnp.concatenate(
        [targets, jnp.zeros((T_pad - T, targets.shape[1]), jnp.float32)],
        axis=0)
    scores = _scores(node_emb, targets_pad)[:, :T]
    padded_logits = scores.reshape(1, -1)

    power = _power_head(node_emb, veh_emb, P['pw1'], P['pb1'],
                        P['pw2'], P['pb2'])
    return (padded_logits, None, power)


# speedup vs baseline: 1.1176x; 1.1176x over previous
import functools

import jax
import jax.numpy as jnp
from jax import lax
from jax.experimental import pallas as pl
from jax.experimental.pallas import tpu as pltpu
from jax.experimental.pallas import tpu_sc as plsc

NC, NS, LANES = 2, 16, 16
NW = NC * NS
K_EDGE = 64
WP = 128
CH = 64


def _mm_bias(x, w, b):
    M, _ = x.shape
    N = w.shape[1]

    def body(x_ref, w_ref, b_ref, o_ref):
        o_ref[...] = jnp.dot(x_ref[...], w_ref[...],
                             preferred_element_type=jnp.float32) + b_ref[...]

    return pl.pallas_call(
        body, out_shape=jax.ShapeDtypeStruct((M, N), jnp.float32),
    )(x, w, b.reshape(1, N))


def _self_init(xl, xr, att, H, ea=None, we=None):
    n, W = xl.shape
    has_ea = ea is not None

    def body(*refs):
        if has_ea:
            xl_ref, xr_ref, att_ref, ea_ref, we_ref, o_ref = refs
        else:
            xl_ref, xr_ref, att_ref, o_ref = refs
        t = xl_ref[...] + xr_ref[...]
        if has_ea:
            xe_row = jnp.dot(jnp.mean(ea_ref[...], axis=0, keepdims=True),
                             we_ref[...], preferred_element_type=jnp.float32)
            t = t + xe_row
        t = jnp.maximum(t, 0.2 * t)
        u = t * att_ref[...]
        zpad = jnp.zeros((n, WP - CH - 1), jnp.float32)
        slabs = []
        for h in range(H):
            s = jnp.sum(u[:, h * CH:(h + 1) * CH], axis=1, keepdims=True)
            p = jnp.exp(s)
            slabs.append(jnp.concatenate(
                [p * xl_ref[:, h * CH:(h + 1) * CH], p, zpad], axis=1))
        if H == 1:
            slabs.append(jnp.zeros((n, WP), jnp.float32))
        o_ref[...] = jnp.stack(slabs, axis=0)

    args = [xl, xr, att.reshape(1, W)]
    if has_ea:
        args += [ea, we]
    return pl.pallas_call(
        body, out_shape=jax.ShapeDtypeStruct((2, n, WP), jnp.float32),
    )(*args)


def _combine(parts, Nd, post=None, ln_g=None, ln_b=None):
    out_w = (parts[0][2] * CH) if parts[0][3] else CH
    has_ln = post in ('ln', 'ln_elu')

    def body(*refs):
        refs = list(refs)
        o = None
        for (_, _, H, concat) in parts:
            p_ref, b_ref = refs.pop(0), refs.pop(0)
            if H == 2:
                outs = [p_ref[h, :Nd, :CH]
                        / (p_ref[h, :Nd, CH:CH + 1] + 1e-16)
                        for h in range(2)]
                v = jnp.concatenate(outs, axis=1) if concat else (
                    (outs[0] + outs[1]) * 0.5)
            else:
                acc = p_ref[0] + p_ref[1]
                v = acc[:Nd, :CH] / (acc[:Nd, CH:CH + 1] + 1e-16)
            v = v + b_ref[...]
            o = v if o is None else o + v
        if has_ln:
            g_ref, lb_ref = refs.pop(0), refs.pop(0)
            mu = jnp.mean(o)
            var = jnp.mean((o - mu) ** 2)
            o = (o - mu) / jnp.sqrt(var + 1e-5) * g_ref[...] + lb_ref[...]
        if post in ('elu', 'ln_elu'):
            o = jnp.where(o > 0, o, jnp.exp(o) - 1.0)
        refs[0][...] = o

    args = []
    for (partials, b, H, concat) in parts:
        args += [partials, b.reshape(1, out_w)]
    if has_ln:
        args += [ln_g.reshape(1, out_w), ln_b.reshape(1, out_w)]
    return pl.pallas_call(
        body, out_shape=jax.ShapeDtypeStruct((Nd, out_w), jnp.float32),
    )(*args)


def _elu(x, x2=None):
    two = x2 is not None

    def body(*refs):
        if two:
            x_ref, y_ref, o_ref = refs
            v = x_ref[...] + y_ref[...]
        else:
            x_ref, o_ref = refs
            v = x_ref[...]
        o_ref[...] = jnp.where(v > 0, v, jnp.exp(v) - 1.0)

    args = [x] if not two else [x, x2]
    return pl.pallas_call(
        body, out_shape=jax.ShapeDtypeStruct(x.shape, jnp.float32),
    )(*args)


def _ln_graph(x, g, b, do_elu=False, x2=None):
    two = x2 is not None
    n, f = x.shape

    def body(*refs):
        if two:
            x_ref, y_ref, g_ref, b_ref, o_ref = refs
            v = x_ref[...] + y_ref[...]
        else:
            x_ref, g_ref, b_ref, o_ref = refs
            v = x_ref[...]
        mu = jnp.mean(v)
        var = jnp.mean((v - mu) ** 2)
        y = (v - mu) / jnp.sqrt(var + 1e-5) * g_ref[...] + b_ref[...]
        if do_elu:
            y = jnp.where(y > 0, y, jnp.exp(y) - 1.0)
        o_ref[...] = y

    args = ([x, x2] if two else [x]) + [g.reshape(1, f), b.reshape(1, f)]
    return pl.pallas_call(
        body, out_shape=jax.ShapeDtypeStruct((n, f), jnp.float32),
    )(*args)


def _scores(node_emb, targets_pad):
    M, Kd = node_emb.shape
    T = targets_pad.shape[0]
    BM = 2000

    def body(x_ref, t_ref, o_ref):
        o_ref[...] = lax.dot_general(
            x_ref[...], t_ref[...], (((1,), (1,)), ((), ())),
            preferred_element_type=jnp.float32) * 0.125

    return pl.pallas_call(
        body,
        grid=(M // BM,),
        in_specs=[pl.BlockSpec((BM, Kd), lambda i: (i, 0)),
                  pl.BlockSpec((T, Kd), lambda i: (0, 0))],
        out_specs=pl.BlockSpec((BM, T), lambda i: (i, 0)),
        out_shape=jax.ShapeDtypeStruct((M, T), jnp.float32),
    )(node_emb, targets_pad)


def _power_head(node_emb, veh_emb, pw1, pb1, pw2, pb2):
    def body(n_ref, v_ref, w1_ref, b1_ref, w2_ref, b2_ref, o_ref):
        ge = jnp.mean(n_ref[...], axis=0, keepdims=True)
        sv = jnp.mean(v_ref[...], axis=0, keepdims=True)
        pf = jnp.concatenate([ge, sv], axis=1)
        hp = jnp.maximum(
            jnp.dot(pf, w1_ref[...], preferred_element_type=jnp.float32)
            + b1_ref[...], 0.0)
        z = jnp.dot(hp, w2_ref[...],
                    preferred_element_type=jnp.float32) + b2_ref[...]
        o_ref[...] = 1.0 / (1.0 + jnp.exp(-z))

    return pl.pallas_call(
        body, out_shape=jax.ShapeDtypeStruct((1, 1), jnp.float32),
    )(node_emb, veh_emb, pw1, pb1.reshape(1, -1), pw2, pb2.reshape(1, -1))


@functools.lru_cache(maxsize=None)
def _edge_sweep_builder(E_pad, Nd_pad, H, has_xe):
    K = K_EDGE
    n_workers = NS if H == 2 else NW
    blocks_per_w = E_pad // (K * n_workers)
    assert blocks_per_w * K * n_workers == E_pad
    assert blocks_per_w % 2 == 0
    mesh = plsc.VectorSubcoreMesh(core_axis_name="c", subcore_axis_name="s",
                                  num_cores=NC, num_subcores=NS)

    def buf2(shape, dtype):
        return [pltpu.VMEM(shape, dtype), pltpu.VMEM(shape, dtype)]

    scratch = (
        buf2((K,), jnp.int32)
        + buf2((K,), jnp.int32)
        + buf2((K,), jnp.int32)
        + buf2((K, CH), jnp.float32)
        + buf2((K, CH), jnp.float32)
        + (buf2((K, CH), jnp.float32) if has_xe else [])
        + buf2((K, WP), jnp.float32)
        + [
            pltpu.VMEM((CH, 16), jnp.float32),
            pltpu.VMEM((2, CH, 16), jnp.float32),
            pltpu.VMEM_SHARED((Nd_pad, WP), jnp.float32),
            pltpu.SemaphoreType.DMA, pltpu.SemaphoreType.DMA,
            pltpu.SemaphoreType.DMA, pltpu.SemaphoreType.DMA,
        ]
    )

    @functools.partial(
        pl.kernel, mesh=mesh,
        compiler_params=pltpu.CompilerParams(needs_layout_passes=False,
                                             use_tc_tiling_on_sc=False),
        out_type=jax.ShapeDtypeStruct((NC, Nd_pad, WP), jnp.float32),
        scratch_types=scratch)
    def k(*refs):
        nx = 3 if has_xe else 2
        tabs = refs[:nx * H]
        src_hbm, dst_hbm = refs[nx * H], refs[nx * H + 1]
        atts = refs[nx * H + 2: nx * H + 2 + H]
        init_hbm = refs[nx * H + 2 + H]
        out_hbm = refs[nx * H + 3 + H]
        it = iter(refs[nx * H + 4 + H:])

        def take2():
            return [next(it), next(it)]

        src_v, dst_v, dsts_v = take2(), take2(), take2()
        xl_v, xr_v = take2(), take2()
        xe_v = take2() if has_xe else [None, None]
        out_v = take2()
        att_v, xlt_v, acc = next(it), next(it), next(it)
        sem_g, sem_s = take2(), take2()

        cid = lax.axis_index("c")
        sid = lax.axis_index("s")

        @pl.when(sid == 0)
        def _():
            pltpu.sync_copy(init_hbm.at[cid], acc)

        lanes = lax.iota(jnp.int32, 16)
        zeros16 = jnp.zeros((16,), jnp.float32)

        def zinit(i, c0):
            for cj in range(CH, WP, 16):
                out_v[0][i, pl.ds(cj, 16)] = zeros16
                out_v[1][i, pl.ds(cj, 16)] = zeros16
            return c0
        lax.fori_loop(0, K, zinit, 0)

        plsc.subcore_barrier()

        def run_edges(xl_hbm, xr_hbm, xe_hbm, att_hbm, worker):
            pltpu.sync_copy(att_hbm, att_v)
            wbase = worker * blocks_per_w

            def issue(b, g):
                base = pl.multiple_of((wbase + g) * K, K)
                pltpu.sync_copy(src_hbm.at[pl.ds(base, K)], src_v[b])
                pltpu.sync_copy(dst_hbm.at[pl.ds(base, K)], dst_v[b])
                pltpu.async_copy(xl_hbm.at[src_v[b]], xl_v[b], sem_g[b])
                pltpu.async_copy(xr_hbm.at[dst_v[b]], xr_v[b], sem_g[b])
                if xe_hbm is not None:
                    pltpu.async_copy(xe_hbm.at[pl.ds(base, K)], xe_v[b],
                                     sem_g[b])

            def compute(b, g):
                pltpu.make_async_copy(xl_hbm.at[src_v[b]], xl_v[b],
                                      sem_g[b]).wait()
                pltpu.make_async_copy(xr_hbm.at[dst_v[b]], xr_v[b],
                                      sem_g[b]).wait()
                if xe_hbm is not None:
                    pltpu.make_async_copy(xe_hbm.at[pl.ds(0, K)], xe_v[b],
                                          sem_g[b]).wait()

                @pl.when(g > 1)
                def _():
                    pltpu.make_async_copy(out_v[b], acc.at[dsts_v[b]],
                                          sem_s[b]).wait()

                def group_body(g2, c2):
                    rows = g2 * 16 + lanes
                    lacc = [zeros16, zeros16, zeros16, zeros16]
                    for c in range(CH):
                        cvec = (lanes + c) & (CH - 1)
                        a = plsc.load_gather(xl_v[b], [rows, cvec])
                        t = a + plsc.load_gather(xr_v[b], [rows, cvec])
                        if xe_hbm is not None:
                            t = t + plsc.load_gather(xe_v[b], [rows, cvec])
                        t = jnp.maximum(t, 0.2 * t)
                        lacc[c % 4] = lacc[c % 4] + att_v[c, :] * t
                        xlt_v[0, c, :] = a
                    pv = jnp.exp((lacc[0] + lacc[1]) + (lacc[2] + lacc[3]))
                    for c in range(CH):
                        cvec = (lanes + c) & (CH - 1)
                        plsc.store_scatter(out_v[b], [rows, cvec],
                                           pv * xlt_v[0, c, :])
                    plsc.store_scatter(
                        out_v[b], [rows, jnp.full((16,), CH, jnp.int32)], pv)
                    return c2

                lax.fori_loop(0, K // 16, group_body, 0)
                for q in range(K // 16):
                    dsts_v[b][pl.ds(q * 16, 16)] = dst_v[b][pl.ds(q * 16, 16)]
                pltpu.async_copy(out_v[b], acc.at[dsts_v[b]], sem_s[b],
                                 add=True)

            issue(0, 0)

            def pair_body(i, carry):
                g0 = i * 2
                issue(1, g0 + 1)
                compute(0, g0)

                @pl.when(i + 1 < blocks_per_w // 2)
                def _():
                    issue(0, g0 + 2)
                compute(1, g0 + 1)
                return carry

            lax.fori_loop(0, blocks_per_w // 2, pair_body, 0)
            pltpu.make_async_copy(out_v[0], acc.at[dsts_v[0]],
                                  sem_s[0]).wait()
            pltpu.make_async_copy(out_v[1], acc.at[dsts_v[1]],
                                  sem_s[1]).wait()

        if H == 2:
            @pl.when(cid == 0)
            def _():
                run_edges(tabs[0], tabs[1], tabs[2] if has_xe else None,
                          atts[0], sid)

            @pl.when(cid == 1)
            def _():
                run_edges(tabs[nx], tabs[nx + 1],
                          tabs[nx + 2] if has_xe else None, atts[1], sid)
        else:
            run_edges(tabs[0], tabs[1], tabs[2] if has_xe else None,
                      atts[0], sid * NC + cid)

        plsc.subcore_barrier()

        @pl.when(sid == 0)
        def _():
            pltpu.sync_copy(acc, out_hbm.at[cid])

    return k


def _round_up(x, m):
    return (x + m - 1) // m * m


def _pad_rows(x, rows):
    r = x.shape[0]
    if r < rows:
        x = jnp.concatenate(
            [x, jnp.zeros((rows - r,) + x.shape[1:], x.dtype)], axis=0)
    return x


def _gat_layer(x_src, x_dst, ei, ea, p, H, self_loops):
    W = H * CH
    Nd = x_dst.shape[0]
    Nd_pad = _round_up(Nd + 1, 16)
    E = ei.shape[1]
    E_pad = _round_up(E, K_EDGE * NW * 2)

    if x_src is x_dst:
        wlr = jnp.concatenate([p['Wl'], p['Wr']], axis=1)
        blr = jnp.concatenate([p['bl'], p['br']])
        xlr = _mm_bias(x_src, wlr, blr)
        xl, xr = xlr[:, :W], xlr[:, W:]
    else:
        xl = _mm_bias(x_src, p['Wl'], p['bl'])
        xr = _mm_bias(x_dst, p['Wr'], p['br'])
    att = p['att'].reshape(-1)

    src = jnp.concatenate(
        [ei[0], jnp.zeros((E_pad - E,), jnp.int32)]) if E_pad > E else ei[0]
    dst = jnp.concatenate(
        [ei[1], jnp.full((E_pad - E,), Nd, jnp.int32)]) if E_pad > E else ei[1]
    xr_pad = _pad_rows(xr, Nd_pad)

    if self_loops:
        init = _self_init(xl, xr, att, H, ea=ea, we=p.get('We'))
        init = jnp.concatenate(
            [init, jnp.zeros((2, Nd_pad - Nd, WP), jnp.float32)], axis=1)
    else:
        init = jnp.zeros((2, Nd_pad, WP), jnp.float32)

    has_xe = ea is not None
    if has_xe:
        ea_pad = (jnp.concatenate(
            [ea, jnp.zeros((E_pad - E, ea.shape[1]), jnp.float32)], axis=0)
            if E_pad > E else ea)
        xe = _mm_bias(ea_pad, p['We'], jnp.zeros((W,), jnp.float32))

    args = []
    for h in range(H):
        args.append(xl[:, h * CH:(h + 1) * CH])
        args.append(xr_pad[:, h * CH:(h + 1) * CH])
        if has_xe:
            args.append(xe[:, h * CH:(h + 1) * CH])
    args += [src, dst]
    diag = (jnp.arange(CH)[:, None] + jnp.arange(16)[None, :]) % CH
    for h in range(H):
        ah = att[h * CH:(h + 1) * CH]
        args.append(ah[diag])
    args.append(init)

    sweep = _edge_sweep_builder(E_pad, Nd_pad, H, has_xe)
    return sweep(*args)


def kernel(dag_x, dag_edge_index, veh_x, rsu_x, v2v_edge_index, v2v_edge_attr,
           v2i_edge_index, v2i_edge_attr, i2v_edge_index, i2v_edge_attr,
           params):
    P = params

    pg1 = _gat_layer(dag_x, dag_x, dag_edge_index, None, P['g1'], 2, True)
    h = _combine([(pg1, P['g1']['b'], 2, True)], dag_x.shape[0],
                 post='ln_elu', ln_g=P['ln1_g'], ln_b=P['ln1_b'])
    pg2 = _gat_layer(h, h, dag_edge_index, None, P['g2'], 1, True)
    node_emb = _combine([(pg2, P['g2']['b'], 1, False)], h.shape[0])

    p_v1 = _gat_layer(veh_x, veh_x, v2v_edge_index, v2v_edge_attr,
                      P['t1_v2v'], 2, True)
    p_r1 = _gat_layer(veh_x, rsu_x, v2i_edge_index, v2i_edge_attr,
                      P['t1_v2i'], 2, False)
    p_v1b = _gat_layer(rsu_x, veh_x, i2v_edge_index, i2v_edge_attr,
                       P['t1_i2v'], 2, False)
    veh_h = _combine([(p_v1, P['t1_v2v']['b'], 2, True),
                      (p_v1b, P['t1_i2v']['b'], 2, True)],
                     veh_x.shape[0], post='elu')
    rsu_h = _combine([(p_r1, P['t1_v2i']['b'], 2, True)],
                     rsu_x.shape[0], post='elu')

    p_v2 = _gat_layer(veh_h, veh_h, v2v_edge_index, v2v_edge_attr,
                      P['t2_v2v'], 1, True)
    p_r2 = _gat_layer(veh_h, rsu_h, v2i_edge_index, v2i_edge_attr,
                      P['t2_v2i'], 1, False)
    p_v2b = _gat_layer(rsu_h, veh_h, i2v_edge_index, i2v_edge_attr,
                       P['t2_i2v'], 1, False)
    veh_emb = _combine([(p_v2, P['t2_v2v']['b'], 1, False),
                        (p_v2b, P['t2_i2v']['b'], 1, False)],
                       veh_h.shape[0], post='ln',
                       ln_g=P['lnv_g'], ln_b=P['lnv_b'])
    rsu_emb = _combine([(p_r2, P['t2_v2i']['b'], 1, False)],
                       rsu_h.shape[0], post='ln',
                       ln_g=P['lnr_g'], ln_b=P['lnr_b'])

    targets = jnp.concatenate([rsu_emb, veh_emb], axis=0)
    T = targets.shape[0]
    T_pad = _round_up(T, 128)
    targets_pad = jnp.concatenate(
        [targets, jnp.zeros((T_pad - T, targets.shape[1]), jnp.float32)],
        axis=0)
    scores = _scores(node_emb, targets_pad)[:, :T]
    padded_logits = scores.reshape(1, -1)

    power = _power_head(node_emb, veh_emb, P['pw1'], P['pb1'],
                        P['pw2'], P['pb2'])
    return (padded_logits, None, power)


# speedup vs baseline: 1.1260x; 1.0075x over previous
import functools

import jax
import jax.numpy as jnp
from jax import lax
from jax.experimental import pallas as pl
from jax.experimental.pallas import tpu as pltpu
from jax.experimental.pallas import tpu_sc as plsc

NC, NS, LANES = 2, 16, 16
NW = NC * NS
K_EDGE = 64
WP = 128
CH = 64


def _mm_bias(x, w, b):
    M, _ = x.shape
    N = w.shape[1]

    def body(x_ref, w_ref, b_ref, o_ref):
        o_ref[...] = jnp.dot(x_ref[...], w_ref[...],
                             preferred_element_type=jnp.float32) + b_ref[...]

    return pl.pallas_call(
        body, out_shape=jax.ShapeDtypeStruct((M, N), jnp.float32),
    )(x, w, b.reshape(1, N))


def _self_init(xl, xr, att, H, ea=None, we=None):
    n, W = xl.shape
    has_ea = ea is not None

    def body(*refs):
        if has_ea:
            xl_ref, xr_ref, att_ref, ea_ref, we_ref, o_ref = refs
        else:
            xl_ref, xr_ref, att_ref, o_ref = refs
        t = xl_ref[...] + xr_ref[...]
        if has_ea:
            xe_row = jnp.dot(jnp.mean(ea_ref[...], axis=0, keepdims=True),
                             we_ref[...], preferred_element_type=jnp.float32)
            t = t + xe_row
        t = jnp.maximum(t, 0.2 * t)
        u = t * att_ref[...]
        zpad = jnp.zeros((n, WP - CH - 1), jnp.float32)
        slabs = []
        for h in range(H):
            s = jnp.sum(u[:, h * CH:(h + 1) * CH], axis=1, keepdims=True)
            p = jnp.exp(s)
            slabs.append(jnp.concatenate(
                [p * xl_ref[:, h * CH:(h + 1) * CH], p, zpad], axis=1))
        if H == 1:
            slabs.append(jnp.zeros((n, WP), jnp.float32))
        o_ref[...] = jnp.stack(slabs, axis=0)

    args = [xl, xr, att.reshape(1, W)]
    if has_ea:
        args += [ea, we]
    return pl.pallas_call(
        body, out_shape=jax.ShapeDtypeStruct((2, n, WP), jnp.float32),
    )(*args)


def _combine(parts, Nd, post=None, ln_g=None, ln_b=None):
    out_w = (parts[0][2] * CH) if parts[0][3] else CH
    has_ln = post in ('ln', 'ln_elu')

    def body(*refs):
        refs = list(refs)
        o = None
        for (_, _, H, concat) in parts:
            p_ref, b_ref = refs.pop(0), refs.pop(0)
            if H == 2:
                outs = [p_ref[h, :Nd, :CH]
                        / (p_ref[h, :Nd, CH:CH + 1] + 1e-16)
                        for h in range(2)]
                v = jnp.concatenate(outs, axis=1) if concat else (
                    (outs[0] + outs[1]) * 0.5)
            else:
                acc = p_ref[0] + p_ref[1]
                v = acc[:Nd, :CH] / (acc[:Nd, CH:CH + 1] + 1e-16)
            v = v + b_ref[...]
            o = v if o is None else o + v
        if has_ln:
            g_ref, lb_ref = refs.pop(0), refs.pop(0)
            mu = jnp.mean(o)
            var = jnp.mean((o - mu) ** 2)
            o = (o - mu) / jnp.sqrt(var + 1e-5) * g_ref[...] + lb_ref[...]
        if post in ('elu', 'ln_elu'):
            o = jnp.where(o > 0, o, jnp.exp(o) - 1.0)
        refs[0][...] = o

    args = []
    for (partials, b, H, concat) in parts:
        args += [partials, b.reshape(1, out_w)]
    if has_ln:
        args += [ln_g.reshape(1, out_w), ln_b.reshape(1, out_w)]
    return pl.pallas_call(
        body, out_shape=jax.ShapeDtypeStruct((Nd, out_w), jnp.float32),
    )(*args)


def _elu(x, x2=None):
    two = x2 is not None

    def body(*refs):
        if two:
            x_ref, y_ref, o_ref = refs
            v = x_ref[...] + y_ref[...]
        else:
            x_ref, o_ref = refs
            v = x_ref[...]
        o_ref[...] = jnp.where(v > 0, v, jnp.exp(v) - 1.0)

    args = [x] if not two else [x, x2]
    return pl.pallas_call(
        body, out_shape=jax.ShapeDtypeStruct(x.shape, jnp.float32),
    )(*args)


def _ln_graph(x, g, b, do_elu=False, x2=None):
    two = x2 is not None
    n, f = x.shape

    def body(*refs):
        if two:
            x_ref, y_ref, g_ref, b_ref, o_ref = refs
            v = x_ref[...] + y_ref[...]
        else:
            x_ref, g_ref, b_ref, o_ref = refs
            v = x_ref[...]
        mu = jnp.mean(v)
        var = jnp.mean((v - mu) ** 2)
        y = (v - mu) / jnp.sqrt(var + 1e-5) * g_ref[...] + b_ref[...]
        if do_elu:
            y = jnp.where(y > 0, y, jnp.exp(y) - 1.0)
        o_ref[...] = y

    args = ([x, x2] if two else [x]) + [g.reshape(1, f), b.reshape(1, f)]
    return pl.pallas_call(
        body, out_shape=jax.ShapeDtypeStruct((n, f), jnp.float32),
    )(*args)


def _scores(node_emb, targets_pad):
    M, Kd = node_emb.shape
    T = targets_pad.shape[0]
    BM = 2000

    def body(x_ref, t_ref, o_ref):
        o_ref[...] = lax.dot_general(
            x_ref[...], t_ref[...], (((1,), (1,)), ((), ())),
            preferred_element_type=jnp.float32) * 0.125

    return pl.pallas_call(
        body,
        grid=(M // BM,),
        in_specs=[pl.BlockSpec((BM, Kd), lambda i: (i, 0)),
                  pl.BlockSpec((T, Kd), lambda i: (0, 0))],
        out_specs=pl.BlockSpec((BM, T), lambda i: (i, 0)),
        out_shape=jax.ShapeDtypeStruct((M, T), jnp.float32),
    )(node_emb, targets_pad)


def _power_head(node_emb, veh_emb, pw1, pb1, pw2, pb2):
    def body(n_ref, v_ref, w1_ref, b1_ref, w2_ref, b2_ref, o_ref):
        ge = jnp.mean(n_ref[...], axis=0, keepdims=True)
        sv = jnp.mean(v_ref[...], axis=0, keepdims=True)
        pf = jnp.concatenate([ge, sv], axis=1)
        hp = jnp.maximum(
            jnp.dot(pf, w1_ref[...], preferred_element_type=jnp.float32)
            + b1_ref[...], 0.0)
        z = jnp.dot(hp, w2_ref[...],
                    preferred_element_type=jnp.float32) + b2_ref[...]
        o_ref[...] = 1.0 / (1.0 + jnp.exp(-z))

    return pl.pallas_call(
        body, out_shape=jax.ShapeDtypeStruct((1, 1), jnp.float32),
    )(node_emb, veh_emb, pw1, pb1.reshape(1, -1), pw2, pb2.reshape(1, -1))


@functools.lru_cache(maxsize=None)
def _edge_sweep_builder(E_pad, Nd_pad, H, has_xe):
    K = K_EDGE
    n_workers = NS if H == 2 else NW
    blocks_per_w = E_pad // (K * n_workers)
    assert blocks_per_w * K * n_workers == E_pad
    assert blocks_per_w % 2 == 0
    mesh = plsc.VectorSubcoreMesh(core_axis_name="c", subcore_axis_name="s",
                                  num_cores=NC, num_subcores=NS)

    def buf2(shape, dtype):
        return [pltpu.VMEM(shape, dtype), pltpu.VMEM(shape, dtype)]

    scratch = (
        buf2((K,), jnp.int32)
        + buf2((K,), jnp.int32)
        + buf2((K,), jnp.int32)
        + buf2((K, CH), jnp.float32)
        + buf2((K, CH), jnp.float32)
        + (buf2((K, CH), jnp.float32) if has_xe else [])
        + buf2((K, WP), jnp.float32)
        + [
            pltpu.VMEM((CH, 16), jnp.float32),
            pltpu.VMEM((2, CH, 16), jnp.float32),
            pltpu.VMEM_SHARED((Nd_pad, WP), jnp.float32),
            pltpu.SemaphoreType.DMA, pltpu.SemaphoreType.DMA,
            pltpu.SemaphoreType.DMA, pltpu.SemaphoreType.DMA,
        ]
    )

    @functools.partial(
        pl.kernel, mesh=mesh,
        compiler_params=pltpu.CompilerParams(needs_layout_passes=False,
                                             use_tc_tiling_on_sc=False),
        out_type=jax.ShapeDtypeStruct((NC, Nd_pad, WP), jnp.float32),
        scratch_types=scratch)
    def k(*refs):
        nx = 3 if has_xe else 2
        tabs = refs[:nx * H]
        src_hbm, dst_hbm = refs[nx * H], refs[nx * H + 1]
        atts = refs[nx * H + 2: nx * H + 2 + H]
        init_hbm = refs[nx * H + 2 + H]
        out_hbm = refs[nx * H + 3 + H]
        it = iter(refs[nx * H + 4 + H:])

        def take2():
            return [next(it), next(it)]

        src_v, dst_v, dsts_v = take2(), take2(), take2()
        xl_v, xr_v = take2(), take2()
        xe_v = take2() if has_xe else [None, None]
        out_v = take2()
        att_v, xlt_v, acc = next(it), next(it), next(it)
        sem_g, sem_s = take2(), take2()

        cid = lax.axis_index("c")
        sid = lax.axis_index("s")

        @pl.when(sid == 0)
        def _():
            pltpu.sync_copy(init_hbm.at[cid], acc)

        lanes = lax.iota(jnp.int32, 16)
        zeros16 = jnp.zeros((16,), jnp.float32)

        def zinit(i, c0):
            for cj in range(CH, WP, 16):
                out_v[0][i, pl.ds(cj, 16)] = zeros16
                out_v[1][i, pl.ds(cj, 16)] = zeros16
            return c0
        lax.fori_loop(0, K, zinit, 0)

        plsc.subcore_barrier()

        def run_edges(xl_hbm, xr_hbm, xe_hbm, att_hbm, worker):
            pltpu.sync_copy(att_hbm, att_v)
            wbase = worker * blocks_per_w

            def issue(b, g):
                base = pl.multiple_of((wbase + g) * K, K)
                pltpu.sync_copy(src_hbm.at[pl.ds(base, K)], src_v[b])
                pltpu.sync_copy(dst_hbm.at[pl.ds(base, K)], dst_v[b])
                pltpu.async_copy(xl_hbm.at[src_v[b]], xl_v[b], sem_g[b])
                pltpu.async_copy(xr_hbm.at[dst_v[b]], xr_v[b], sem_g[b])
                if xe_hbm is not None:
                    pltpu.async_copy(xe_hbm.at[pl.ds(base, K)], xe_v[b],
                                     sem_g[b])

            def compute(b, g):
                pltpu.make_async_copy(xl_hbm.at[src_v[b]], xl_v[b],
                                      sem_g[b]).wait()
                pltpu.make_async_copy(xr_hbm.at[dst_v[b]], xr_v[b],
                                      sem_g[b]).wait()
                if xe_hbm is not None:
                    pltpu.make_async_copy(xe_hbm.at[pl.ds(0, K)], xe_v[b],
                                          sem_g[b]).wait()

                @pl.when(g > 1)
                def _():
                    pltpu.make_async_copy(out_v[b], acc.at[dsts_v[b]],
                                          sem_s[b]).wait()

                def group_body(g2, c2):
                    rows = g2 * 16 + lanes
                    lacc = [zeros16] * 8
                    for c in range(CH):
                        cvec = (lanes + c) & (CH - 1)
                        a = plsc.load_gather(xl_v[b], [rows, cvec])
                        t = a + plsc.load_gather(xr_v[b], [rows, cvec])
                        if xe_hbm is not None:
                            t = t + plsc.load_gather(xe_v[b], [rows, cvec])
                        t = jnp.maximum(t, 0.2 * t)
                        lacc[c % 8] = lacc[c % 8] + att_v[c, :] * t
                        xlt_v[0, c, :] = a
                    pv = jnp.exp(((lacc[0] + lacc[1]) + (lacc[2] + lacc[3]))
                                 + ((lacc[4] + lacc[5]) + (lacc[6] + lacc[7])))
                    for c in range(CH):
                        cvec = (lanes + c) & (CH - 1)
                        plsc.store_scatter(out_v[b], [rows, cvec],
                                           pv * xlt_v[0, c, :])
                    plsc.store_scatter(
                        out_v[b], [rows, jnp.full((16,), CH, jnp.int32)], pv)
                    return c2

                lax.fori_loop(0, K // 16, group_body, 0)
                for q in range(K // 16):
                    dsts_v[b][pl.ds(q * 16, 16)] = dst_v[b][pl.ds(q * 16, 16)]
                pltpu.async_copy(out_v[b], acc.at[dsts_v[b]], sem_s[b],
                                 add=True)

            issue(0, 0)

            def pair_body(i, carry):
                g0 = i * 2
                issue(1, g0 + 1)
                compute(0, g0)

                @pl.when(i + 1 < blocks_per_w // 2)
                def _():
                    issue(0, g0 + 2)
                compute(1, g0 + 1)
                return carry

            lax.fori_loop(0, blocks_per_w // 2, pair_body, 0)
            pltpu.make_async_copy(out_v[0], acc.at[dsts_v[0]],
                                  sem_s[0]).wait()
            pltpu.make_async_copy(out_v[1], acc.at[dsts_v[1]],
                                  sem_s[1]).wait()

        if H == 2:
            @pl.when(cid == 0)
            def _():
                run_edges(tabs[0], tabs[1], tabs[2] if has_xe else None,
                          atts[0], sid)

            @pl.when(cid == 1)
            def _():
                run_edges(tabs[nx], tabs[nx + 1],
                          tabs[nx + 2] if has_xe else None, atts[1], sid)
        else:
            run_edges(tabs[0], tabs[1], tabs[2] if has_xe else None,
                      atts[0], sid * NC + cid)

        plsc.subcore_barrier()

        @pl.when(sid == 0)
        def _():
            pltpu.sync_copy(acc, out_hbm.at[cid])

    return k


def _round_up(x, m):
    return (x + m - 1) // m * m


def _pad_rows(x, rows):
    r = x.shape[0]
    if r < rows:
        x = jnp.concatenate(
            [x, jnp.zeros((rows - r,) + x.shape[1:], x.dtype)], axis=0)
    return x


def _gat_layer(x_src, x_dst, ei, ea, p, H, self_loops):
    W = H * CH
    Nd = x_dst.shape[0]
    Nd_pad = _round_up(Nd + 1, 16)
    E = ei.shape[1]
    E_pad = _round_up(E, K_EDGE * NW * 2)

    if x_src is x_dst:
        wlr = jnp.concatenate([p['Wl'], p['Wr']], axis=1)
        blr = jnp.concatenate([p['bl'], p['br']])
        xlr = _mm_bias(x_src, wlr, blr)
        xl, xr = xlr[:, :W], xlr[:, W:]
    else:
        xl = _mm_bias(x_src, p['Wl'], p['bl'])
        xr = _mm_bias(x_dst, p['Wr'], p['br'])
    att = p['att'].reshape(-1)

    src = jnp.concatenate(
        [ei[0], jnp.zeros((E_pad - E,), jnp.int32)]) if E_pad > E else ei[0]
    dst = jnp.concatenate(
        [ei[1], jnp.full((E_pad - E,), Nd, jnp.int32)]) if E_pad > E else ei[1]
    xr_pad = _pad_rows(xr, Nd_pad)

    if self_loops:
        init = _self_init(xl, xr, att, H, ea=ea, we=p.get('We'))
        init = jnp.concatenate(
            [init, jnp.zeros((2, Nd_pad - Nd, WP), jnp.float32)], axis=1)
    else:
        init = jnp.zeros((2, Nd_pad, WP), jnp.float32)

    has_xe = ea is not None
    if has_xe:
        ea_pad = (jnp.concatenate(
            [ea, jnp.zeros((E_pad - E, ea.shape[1]), jnp.float32)], axis=0)
            if E_pad > E else ea)
        xe = _mm_bias(ea_pad, p['We'], jnp.zeros((W,), jnp.float32))

    args = []
    for h in range(H):
        args.append(xl[:, h * CH:(h + 1) * CH])
        args.append(xr_pad[:, h * CH:(h + 1) * CH])
        if has_xe:
            args.append(xe[:, h * CH:(h + 1) * CH])
    args += [src, dst]
    diag = (jnp.arange(CH)[:, None] + jnp.arange(16)[None, :]) % CH
    for h in range(H):
        ah = att[h * CH:(h + 1) * CH]
        args.append(ah[diag])
    args.append(init)

    sweep = _edge_sweep_builder(E_pad, Nd_pad, H, has_xe)
    return sweep(*args)


def kernel(dag_x, dag_edge_index, veh_x, rsu_x, v2v_edge_index, v2v_edge_attr,
           v2i_edge_index, v2i_edge_attr, i2v_edge_index, i2v_edge_attr,
           params):
    P = params

    pg1 = _gat_layer(dag_x, dag_x, dag_edge_index, None, P['g1'], 2, True)
    h = _combine([(pg1, P['g1']['b'], 2, True)], dag_x.shape[0],
                 post='ln_elu', ln_g=P['ln1_g'], ln_b=P['ln1_b'])
    pg2 = _gat_layer(h, h, dag_edge_index, None, P['g2'], 1, True)
    node_emb = _combine([(pg2, P['g2']['b'], 1, False)], h.shape[0])

    p_v1 = _gat_layer(veh_x, veh_x, v2v_edge_index, v2v_edge_attr,
                      P['t1_v2v'], 2, True)
    p_r1 = _gat_layer(veh_x, rsu_x, v2i_edge_index, v2i_edge_attr,
                      P['t1_v2i'], 2, False)
    p_v1b = _gat_layer(rsu_x, veh_x, i2v_edge_index, i2v_edge_attr,
                       P['t1_i2v'], 2, False)
    veh_h = _combine([(p_v1, P['t1_v2v']['b'], 2, True),
                      (p_v1b, P['t1_i2v']['b'], 2, True)],
                     veh_x.shape[0], post='elu')
    rsu_h = _combine([(p_r1, P['t1_v2i']['b'], 2, True)],
                     rsu_x.shape[0], post='elu')

    p_v2 = _gat_layer(veh_h, veh_h, v2v_edge_index, v2v_edge_attr,
                      P['t2_v2v'], 1, True)
    p_r2 = _gat_layer(veh_h, rsu_h, v2i_edge_index, v2i_edge_attr,
                      P['t2_v2i'], 1, False)
    p_v2b = _gat_layer(rsu_h, veh_h, i2v_edge_index, i2v_edge_attr,
                       P['t2_i2v'], 1, False)
    veh_emb = _combine([(p_v2, P['t2_v2v']['b'], 1, False),
                        (p_v2b, P['t2_i2v']['b'], 1, False)],
                       veh_h.shape[0], post='ln',
                       ln_g=P['lnv_g'], ln_b=P['lnv_b'])
    rsu_emb = _combine([(p_r2, P['t2_v2i']['b'], 1, False)],
                       rsu_h.shape[0], post='ln',
                       ln_g=P['lnr_g'], ln_b=P['lnr_b'])

    targets = jnp.concatenate([rsu_emb, veh_emb], axis=0)
    T = targets.shape[0]
    T_pad = _round_up(T, 128)
    targets_pad = jnp.concatenate(
        [targets, jnp.zeros((T_pad - T, targets.shape[1]), jnp.float32)],
        axis=0)
    scores = _scores(node_emb, targets_pad)[:, :T]
    padded_logits = scores.reshape(1, -1)

    power = _power_head(node_emb, veh_emb, P['pw1'], P['pb1'],
                        P['pw2'], P['pb2'])
    return (padded_logits, None, power)
